# baseline (speedup 1.0000x reference)
import functools

import jax
import jax.numpy as jnp
from jax import lax
from jax.experimental import pallas as pl
from jax.experimental.pallas import tpu as pltpu
from jax.experimental.pallas import tpu_sc as plsc

N = 50000
E = 800000
C = 4096
H = 64
NPAD = 50048
RPT = 3128
EB = 1000
EX = 896000
R = 3128
NG = NPAD // R

_mesh = plsc.VectorSubcoreMesh(core_axis_name="c", subcore_axis_name="s")

_f32 = jnp.float32
_i32 = jnp.int32


def _fill_const_2d(buf, nrows, width, val):
    vec = jnp.full((16,), val, _f32)

    def body(i, carry):
        for w0 in range(0, width, 16):
            buf[i, pl.ds(w0, 16)] = vec
        return carry

    lax.fori_loop(0, nrows, body, 0)


def _fill_const_1d(buf, n, val):
    vec = jnp.full((16,), val, _f32)

    def body(i, carry):
        buf[pl.ds(i * 16, 16)] = vec
        return carry

    lax.fori_loop(0, n // 16, body, 0)
    if n % 16:
        buf[pl.ds(n - 16, 16)] = vec


def _zero_rows_2d(agg, zbuf, row0):
    for off in (0, 1024, 2048):
        pltpu.sync_copy(zbuf.at[:, :], agg.at[pl.ds(row0 + off, 1024), :])
    pltpu.sync_copy(zbuf.at[pl.ds(0, 56), :], agg.at[pl.ds(row0 + 3072, 56), :])


def _zero_rows_1d(agg, zbuf, row0):
    for off in (0, 1024, 2048):
        pltpu.sync_copy(zbuf.at[pl.ds(0, 1024)], agg.at[pl.ds(row0 + off, 1024)])
    pltpu.sync_copy(zbuf.at[pl.ds(0, 56)], agg.at[pl.ds(row0 + 3072, 56)])


_CHUNKS = ((0, 1024), (1024, 1024), (2048, 1024), (3072, 56))


def _writeout_2d(agg, buf, out, c, row0):
    for off, sz in _CHUNKS:
        pltpu.sync_copy(agg.at[pl.ds(row0 + off, sz), :], buf.at[pl.ds(0, sz), :])
        pltpu.sync_copy(buf.at[pl.ds(0, sz), :], out.at[c, pl.ds(row0 + off, sz), :])


def _segsum_edges(src, dst, z, agg, src_v, dst_v, rows_v, sem, base, nblocks):
    def body(i, carry):
        b = base + i * EB
        pltpu.sync_copy(src.at[pl.ds(b, EB)], src_v)
        pltpu.sync_copy(dst.at[pl.ds(b, EB)], dst_v)
        pltpu.async_copy(z.at[src_v], rows_v, sem).wait()
        pltpu.sync_copy(rows_v, agg.at[dst_v], add=True)
        return carry

    lax.fori_loop(0, nblocks, body, 0)


@functools.partial(
    pl.kernel,
    out_type=jax.ShapeDtypeStruct((2, NPAD), _f32),
    mesh=_mesh,
    compiler_params=pltpu.CompilerParams(use_tc_tiling_on_sc=False),
    scratch_types=[
        pltpu.VMEM((EB,), _i32),
        pltpu.VMEM((EB,), _f32),
        pltpu.VMEM((1024,), _f32),
        pltpu.VMEM_SHARED((NPAD,), _f32),
    ],
)
def _sc_degree(dst, out, dst_v, ones_v, zbuf, deg_sh):
    c = lax.axis_index("c")
    s = lax.axis_index("s")
    row0 = s * RPT
    _fill_const_1d(zbuf, 1024, 0.0)
    _fill_const_1d(ones_v, EB, 1.0)
    _zero_rows_1d(deg_sh, zbuf, row0)
    plsc.subcore_barrier()

    base = c * 400000 + s * 25000

    def body(i, carry):
        pltpu.sync_copy(dst.at[pl.ds(base + i * EB, EB)], dst_v)
        pltpu.sync_copy(ones_v.at[pl.ds(0, EB)], deg_sh.at[dst_v], add=True)
        return carry

    lax.fori_loop(0, 25, body, 0)
    plsc.subcore_barrier()

    for off, sz in _CHUNKS:
        pltpu.sync_copy(deg_sh.at[pl.ds(row0 + off, sz)], zbuf.at[pl.ds(0, sz)])
        pltpu.sync_copy(zbuf.at[pl.ds(0, sz)], out.at[c, pl.ds(row0 + off, sz)])


@functools.partial(
    pl.kernel,
    out_type=jax.ShapeDtypeStruct((4, NPAD, 16), _f32),
    mesh=_mesh,
    compiler_params=pltpu.CompilerParams(use_tc_tiling_on_sc=False),
    scratch_types=[
        pltpu.VMEM((EB,), _i32),
        pltpu.VMEM((EB,), _i32),
        pltpu.VMEM((EB, 16), _f32),
        pltpu.VMEM((1024, 16), _f32),
        pltpu.VMEM((1024, 16), _f32),
        pltpu.VMEM_SHARED((NPAD, 16), _f32),
        pltpu.SemaphoreType.DMA,
    ],
)
def _sc_segsum64(src, dst, z4, out, src_v, dst_v, rows_v, zbuf, wbuf, agg, sem):
    c = lax.axis_index("c")
    s = lax.axis_index("s")
    row0 = s * RPT
    _fill_const_2d(zbuf, 1024, 16, 0.0)
    _zero_rows_2d(agg, zbuf, row0)

    base = s * (EX // 16)
    for p in range(2):
        chunk = 2 * c + p
        plsc.subcore_barrier()
        _segsum_edges(src, dst, z4.at[chunk], agg, src_v, dst_v, rows_v, sem,
                      base, EX // 16 // EB)
        plsc.subcore_barrier()
        for off, sz in _CHUNKS:
            pltpu.sync_copy(agg.at[pl.ds(row0 + off, sz), :],
                            wbuf.at[pl.ds(0, sz), :])
            pltpu.sync_copy(wbuf.at[pl.ds(0, sz), :],
                            out.at[chunk, pl.ds(row0 + off, sz), :])
        if p == 0:
            _zero_rows_2d(agg, zbuf, row0)


@functools.partial(
    pl.kernel,
    out_type=jax.ShapeDtypeStruct((2, C, H), _f32),
    mesh=_mesh,
    compiler_params=pltpu.CompilerParams(use_tc_tiling_on_sc=False),
    scratch_types=[
        pltpu.VMEM((256,), _i32),
        pltpu.VMEM((256, H), _f32),
        pltpu.SemaphoreType.DMA,
    ],
)
def _sc_cand_gather(h, cuv, out, idx_v, rows_v, sem):
    c = lax.axis_index("c")
    s = lax.axis_index("s")
    base = s * 256
    pltpu.sync_copy(cuv.at[c, pl.ds(base, 256)], idx_v)
    pltpu.async_copy(h.at[idx_v], rows_v, sem).wait()
    pltpu.sync_copy(rows_v, out.at[c, pl.ds(base, 256), :])


def _encode_body(kid, other, d0, d1, ktab, w, z4_ref, nrm_ref):
    n = lax.rsqrt(d0[...] + d1[...] + 1.0)
    oh = (kid[...] == lax.broadcasted_iota(_i32, (R, 8), 1)).astype(_f32)
    emb = jnp.dot(oh, ktab[...], preferred_element_type=_f32,
                  precision=lax.Precision.HIGHEST)
    x = jnp.concatenate([emb, other[...], jnp.zeros((R, 3), _f32)], axis=1)
    xw = jnp.dot(x, w[...], preferred_element_type=_f32)
    row = (pl.program_id(0) * R
           + lax.broadcasted_iota(_i32, (R, 1), 0))
    z = xw * (n * (row < N).astype(_f32))
    z4_ref[...] = jnp.stack([z[:, :16], z[:, 16:32], z[:, 32:48], z[:, 48:]],
                            axis=0)
    nrm_ref[...] = n


def _tc_encode(kid2, other, d0, d1, ktab8, w0p):
    return pl.pallas_call(
        _encode_body,
        grid=(NG,),
        in_specs=[
            pl.BlockSpec((R, 1), lambda i: (i, 0)),
            pl.BlockSpec((R, 5), lambda i: (i, 0)),
            pl.BlockSpec((R, 1), lambda i: (i, 0)),
            pl.BlockSpec((R, 1), lambda i: (i, 0)),
            pl.BlockSpec((8, 8), lambda i: (0, 0)),
            pl.BlockSpec((16, H), lambda i: (0, 0)),
        ],
        out_specs=[
            pl.BlockSpec((4, R, 16), lambda i: (0, i, 0)),
            pl.BlockSpec((R, 1), lambda i: (i, 0)),
        ],
        out_shape=[
            jax.ShapeDtypeStruct((4, NPAD, 16), _f32),
            jax.ShapeDtypeStruct((NPAD, 1), _f32),
        ],
    )(kid2, other, d0, d1, ktab8, w0p)


def _combine0_body(s2, nrm, w, b, z4_ref):
    n = nrm[...]
    sblk = s2[...]
    agg = n * (sblk[0] + sblk[1])
    h = jnp.maximum(jnp.dot(agg, w[...], preferred_element_type=_f32, precision=lax.Precision.HIGHEST) + b[...], 0.0)
    row = pl.program_id(0) * R + lax.broadcasted_iota(_i32, (R, 1), 0)
    z = h * (n * (row < N).astype(_f32))
    z4_ref[...] = jnp.stack([z[:, :16], z[:, 16:32], z[:, 32:48], z[:, 48:]],
                            axis=0)


def _tc_combine0(s2, nrm, w0p, b0):
    return pl.pallas_call(
        _combine0_body,
        grid=(NG,),
        in_specs=[
            pl.BlockSpec((2, R, 16), lambda i: (0, i, 0)),
            pl.BlockSpec((R, 1), lambda i: (i, 0)),
            pl.BlockSpec((16, H), lambda i: (0, 0)),
            pl.BlockSpec((1, H), lambda i: (0, 0)),
        ],
        out_specs=pl.BlockSpec((4, R, 16), lambda i: (0, i, 0)),
        out_shape=jax.ShapeDtypeStruct((4, NPAD, 16), _f32),
    )(s2, nrm, w0p, b0)


def _combine_body(s4, nrm, b, w, z4_ref):
    n = nrm[...]
    sblk = s4[...]
    ss = jnp.concatenate([sblk[0], sblk[1], sblk[2], sblk[3]], axis=1)
    h = jnp.maximum(n * ss + b[...], 0.0)
    hw = jnp.dot(h, w[...], preferred_element_type=_f32)
    row = pl.program_id(0) * R + lax.broadcasted_iota(_i32, (R, 1), 0)
    zo = hw * (n * (row < N).astype(_f32))
    z4_ref[...] = jnp.stack([zo[:, :16], zo[:, 16:32], zo[:, 32:48],
                             zo[:, 48:]], axis=0)


def _tc_combine(s4, nrm, b_prev, w_next):
    return pl.pallas_call(
        _combine_body,
        grid=(NG,),
        in_specs=[
            pl.BlockSpec((4, R, 16), lambda i: (0, i, 0)),
            pl.BlockSpec((R, 1), lambda i: (i, 0)),
            pl.BlockSpec((1, H), lambda i: (0, 0)),
            pl.BlockSpec((H, H), lambda i: (0, 0)),
        ],
        out_specs=pl.BlockSpec((4, R, 16), lambda i: (0, i, 0)),
        out_shape=jax.ShapeDtypeStruct((4, NPAD, 16), _f32),
    )(s4, nrm, b_prev, w_next)


def _final_body(s4, nrm, b, h_ref, acc_ref):
    n = nrm[...]
    sblk = s4[...]
    ss = jnp.concatenate([sblk[0], sblk[1], sblk[2], sblk[3]], axis=1)
    h = jnp.maximum(n * ss + b[...], 0.0)
    h_ref[...] = h

    @pl.when(pl.program_id(0) == 0)
    def _():
        acc_ref[...] = jnp.zeros((8, H), _f32)

    row = pl.program_id(0) * R + lax.broadcasted_iota(_i32, (R, 1), 0)
    hm = h * (row < N).astype(_f32)
    acc_ref[...] += jnp.sum(hm.reshape(R // 8, 8, H), axis=0)


def _tc_final(s4, nrm, b):
    return pl.pallas_call(
        _final_body,
        grid=(NG,),
        in_specs=[
            pl.BlockSpec((4, R, 16), lambda i: (0, i, 0)),
            pl.BlockSpec((R, 1), lambda i: (i, 0)),
            pl.BlockSpec((1, H), lambda i: (0, 0)),
        ],
        out_specs=[
            pl.BlockSpec((R, H), lambda i: (i, 0)),
            pl.BlockSpec((8, H), lambda i: (0, 0)),
        ],
        out_shape=[
            jax.ShapeDtypeStruct((NPAD, H), _f32),
            jax.ShapeDtypeStruct((8, H), _f32),
        ],
    )(s4, nrm, b)


def _score_body(u, v, acc, wg, bg, wc1, bc1, w2, bc2, out_ref):
    tot = jnp.sum(acc[...], axis=0, keepdims=True) * (1.0 / N)
    g = jnp.dot(tot, wg[...], preferred_element_type=_f32) + bg[...]
    feat = jnp.concatenate(
        [u[...], v[...], jnp.broadcast_to(g, (C, H))], axis=1)
    hid = jnp.maximum(jnp.dot(feat, wc1[...], preferred_element_type=_f32)
                      + bc1[...], 0.0)
    lg = jnp.dot(hid, w2[...], preferred_element_type=_f32) + bc2[...]
    out_ref[...] = lg


def _tc_score(u, v, acc, wg, bg, wc1, bc1, w2, bc2):
    whole = lambda shp: pl.BlockSpec(shp, lambda: (0, 0))
    return pl.pallas_call(
        _score_body,
        in_specs=[
            whole((C, H)), whole((C, H)), whole((8, H)),
            whole((H, H)), whole((1, H)),
            whole((3 * H, H)), whole((1, H)),
            whole((H, 1)), whole((1, 1)),
        ],
        out_specs=whole((C, 1)),
        out_shape=jax.ShapeDtypeStruct((C, 1), _f32),
    )(u, v, acc, wg, bg, wc1, bc1, w2, bc2)


def kernel(kind_ids, other_feats, edge_index, cand_u, cand_v, kind_table,
           W0, b0, W1, b1, W2, b2, Wg, bg, Wc1, bc1, Wc2, bc2):
    ei = edge_index.astype(_i32)
    loop = jnp.arange(N, dtype=_i32)
    nd = EX - E - N
    src = jnp.concatenate([ei[0], loop, jnp.full((nd,), N, _i32)])
    dst = jnp.concatenate([ei[1], loop, jnp.arange(nd, dtype=_i32) % N])
    deg2 = _sc_degree(dst)
    ktab8 = jnp.concatenate([kind_table, jnp.zeros((2, 8), _f32)], axis=0)
    kid_p = jnp.pad(kind_ids.astype(_i32), (0, NPAD - N)).reshape(NPAD, 1)
    of_p = jnp.pad(other_feats, ((0, NPAD - N), (0, 0)))
    w0p = jnp.concatenate([W0, jnp.zeros((3, H), _f32)], axis=0)
    z0, nrm = _tc_encode(kid_p, of_p, deg2[0].reshape(NPAD, 1),
                         deg2[1].reshape(NPAD, 1), ktab8, w0p)
    s0 = _sc_segsum64(src, dst, z0)
    z1 = _tc_combine(s0, nrm, b0.reshape(1, H), W1)
    s1 = _sc_segsum64(src, dst, z1)
    z2 = _tc_combine(s1, nrm, b1.reshape(1, H), W2)
    s2 = _sc_segsum64(src, dst, z2)
    h3, acc = _tc_final(s2, nrm, b2.reshape(1, H))
    cuv = jnp.stack([cand_u.astype(_i32), cand_v.astype(_i32)], axis=0)
    uvr = _sc_cand_gather(h3, cuv)
    ur, vr = uvr[0], uvr[1]
    lg = _tc_score(ur, vr, acc, Wg, bg.reshape(1, H), Wc1,
                   bc1.reshape(1, H), Wc2, bc2.reshape(1, 1))
    return lg.reshape(C)


# speedup vs baseline: 1.0001x; 1.0001x over previous
import functools

import jax
import jax.numpy as jnp
from jax import lax
from jax.experimental import pallas as pl
from jax.experimental.pallas import tpu as pltpu
from jax.experimental.pallas import tpu_sc as plsc

N = 50000
E = 800000
C = 4096
H = 64
NPAD = 50048
RPT = 3128
EB = 1000
EX = 896000
R = 3128
NG = NPAD // R

_mesh = plsc.VectorSubcoreMesh(core_axis_name="c", subcore_axis_name="s")

_f32 = jnp.float32
_i32 = jnp.int32


def _fill_const_2d(buf, nrows, width, val):
    vec = jnp.full((16,), val, _f32)

    def body(i, carry):
        for w0 in range(0, width, 16):
            buf[i, pl.ds(w0, 16)] = vec
        return carry

    lax.fori_loop(0, nrows, body, 0)


def _fill_const_1d(buf, n, val):
    vec = jnp.full((16,), val, _f32)

    def body(i, carry):
        buf[pl.ds(i * 16, 16)] = vec
        return carry

    lax.fori_loop(0, n // 16, body, 0)
    if n % 16:
        buf[pl.ds(n - 16, 16)] = vec


def _zero_rows_2d(agg, zbuf, row0):
    for off in (0, 1024, 2048):
        pltpu.sync_copy(zbuf.at[:, :], agg.at[pl.ds(row0 + off, 1024), :])
    pltpu.sync_copy(zbuf.at[pl.ds(0, 56), :], agg.at[pl.ds(row0 + 3072, 56), :])


def _zero_rows_1d(agg, zbuf, row0):
    for off in (0, 1024, 2048):
        pltpu.sync_copy(zbuf.at[pl.ds(0, 1024)], agg.at[pl.ds(row0 + off, 1024)])
    pltpu.sync_copy(zbuf.at[pl.ds(0, 56)], agg.at[pl.ds(row0 + 3072, 56)])


_CHUNKS = ((0, 1024), (1024, 1024), (2048, 1024), (3072, 56))


def _writeout_2d(agg, buf, out, c, row0):
    for off, sz in _CHUNKS:
        pltpu.sync_copy(agg.at[pl.ds(row0 + off, sz), :], buf.at[pl.ds(0, sz), :])
        pltpu.sync_copy(buf.at[pl.ds(0, sz), :], out.at[c, pl.ds(row0 + off, sz), :])


def _issue_idx(src, dst, sv, dv, sem, b):
    pltpu.async_copy(src.at[pl.ds(b, EB)], sv, sem)
    pltpu.async_copy(dst.at[pl.ds(b, EB)], dv, sem)


def _wait_idx(src, sv, dv, sem):
    pltpu.make_async_copy(src.at[pl.ds(0, EB)], sv, sem).wait()
    pltpu.make_async_copy(src.at[pl.ds(0, EB)], dv, sem).wait()


def _segsum_edges_pipe(src, dst, z, agg, sv, dv, rv, si, sg, ss, base, nb):
    _issue_idx(src, dst, sv[0], dv[0], si[0], base)
    _issue_idx(src, dst, sv[1], dv[1], si[1], base + EB)
    _wait_idx(src, sv[0], dv[0], si[0])
    pltpu.async_copy(z.at[sv[0]], rv[0], sg[0])

    def quad(k, carry):
        for b4 in range(4):
            g = 4 * k + b4
            rb = b4 % 2
            rb2 = 1 - rb
            j1 = (b4 + 1) % 4
            j2 = (b4 + 2) % 4

            @pl.when(g >= 1)
            def _():
                pltpu.make_async_copy(rv[rb2], agg.at[dv[0]], ss[rb2]).wait()

            @pl.when(g + 1 < nb)
            def _():
                _wait_idx(src, sv[j1], dv[j1], si[j1])
                pltpu.async_copy(z.at[sv[j1]], rv[rb2], sg[rb2])

            pltpu.make_async_copy(z.at[sv[0]], rv[rb], sg[rb]).wait()

            @pl.when(g < nb - 1)
            def _():
                pltpu.async_copy(rv[rb], agg.at[dv[b4]], ss[rb], add=True)

            @pl.when(g == nb - 1)
            def _():
                pltpu.sync_copy(rv[rb], agg.at[dv[b4]], add=True)

            @pl.when(g + 2 < nb)
            def _():
                _issue_idx(src, dst, sv[j2], dv[j2], si[j2],
                           base + (g + 2) * EB)
        return carry

    lax.fori_loop(0, nb // 4, quad, 0)


@functools.partial(
    pl.kernel,
    out_type=jax.ShapeDtypeStruct((2, NPAD), _f32),
    mesh=_mesh,
    compiler_params=pltpu.CompilerParams(use_tc_tiling_on_sc=False),
    scratch_types=[
        pltpu.VMEM((EB,), _i32),
        pltpu.VMEM((EB,), _f32),
        pltpu.VMEM((1024,), _f32),
        pltpu.VMEM_SHARED((NPAD,), _f32),
    ],
)
def _sc_degree(dst, out, dst_v, ones_v, zbuf, deg_sh):
    c = lax.axis_index("c")
    s = lax.axis_index("s")
    row0 = s * RPT
    _fill_const_1d(zbuf, 1024, 0.0)
    _fill_const_1d(ones_v, EB, 1.0)
    _zero_rows_1d(deg_sh, zbuf, row0)
    plsc.subcore_barrier()

    base = c * 400000 + s * 25000

    def body(i, carry):
        pltpu.sync_copy(dst.at[pl.ds(base + i * EB, EB)], dst_v)
        pltpu.sync_copy(ones_v.at[pl.ds(0, EB)], deg_sh.at[dst_v], add=True)
        return carry

    lax.fori_loop(0, 25, body, 0)
    plsc.subcore_barrier()

    for off, sz in _CHUNKS:
        pltpu.sync_copy(deg_sh.at[pl.ds(row0 + off, sz)], zbuf.at[pl.ds(0, sz)])
        pltpu.sync_copy(zbuf.at[pl.ds(0, sz)], out.at[c, pl.ds(row0 + off, sz)])


@functools.partial(
    pl.kernel,
    out_type=jax.ShapeDtypeStruct((4, NPAD, 16), _f32),
    mesh=_mesh,
    compiler_params=pltpu.CompilerParams(use_tc_tiling_on_sc=False),
    scratch_types=[
        pltpu.VMEM((EB,), _i32), pltpu.VMEM((EB,), _i32),
        pltpu.VMEM((EB,), _i32), pltpu.VMEM((EB,), _i32),
        pltpu.VMEM((EB,), _i32), pltpu.VMEM((EB,), _i32),
        pltpu.VMEM((EB,), _i32), pltpu.VMEM((EB,), _i32),
        pltpu.VMEM((EB, 16), _f32), pltpu.VMEM((EB, 16), _f32),
        pltpu.VMEM((1024, 16), _f32),
        pltpu.VMEM((1024, 16), _f32),
        pltpu.VMEM_SHARED((NPAD, 16), _f32),
        pltpu.SemaphoreType.DMA, pltpu.SemaphoreType.DMA,
        pltpu.SemaphoreType.DMA, pltpu.SemaphoreType.DMA,
        pltpu.SemaphoreType.DMA, pltpu.SemaphoreType.DMA,
        pltpu.SemaphoreType.DMA, pltpu.SemaphoreType.DMA,
    ],
)
def _sc_segsum64(src, dst, z4, out, sv0, sv1, sv2, sv3, dv0, dv1, dv2, dv3,
                 rv0, rv1, zbuf, wbuf, agg,
                 si0, si1, si2, si3, sg0, sg1, ss0, ss1):
    c = lax.axis_index("c")
    s = lax.axis_index("s")
    row0 = s * RPT
    sv = [sv0, sv1, sv2, sv3]
    dv = [dv0, dv1, dv2, dv3]
    rv = [rv0, rv1]
    si = [si0, si1, si2, si3]
    sg = [sg0, sg1]
    ss = [ss0, ss1]
    _fill_const_2d(zbuf, 1024, 16, 0.0)
    _zero_rows_2d(agg, zbuf, row0)

    base = s * (EX // 16)
    nb = EX // 16 // EB
    for p in range(2):
        chunk = 2 * c + p
        plsc.subcore_barrier()
        _segsum_edges_pipe(src, dst, z4.at[chunk], agg, sv, dv, rv,
                           si, sg, ss, base, nb)
        plsc.subcore_barrier()
        for off, sz in _CHUNKS:
            pltpu.sync_copy(agg.at[pl.ds(row0 + off, sz), :],
                            wbuf.at[pl.ds(0, sz), :])
            pltpu.sync_copy(wbuf.at[pl.ds(0, sz), :],
                            out.at[chunk, pl.ds(row0 + off, sz), :])
        if p == 0:
            _zero_rows_2d(agg, zbuf, row0)


@functools.partial(
    pl.kernel,
    out_type=jax.ShapeDtypeStruct((2, C, H), _f32),
    mesh=_mesh,
    compiler_params=pltpu.CompilerParams(use_tc_tiling_on_sc=False),
    scratch_types=[
        pltpu.VMEM((256,), _i32),
        pltpu.VMEM((256, H), _f32),
        pltpu.SemaphoreType.DMA,
    ],
)
def _sc_cand_gather(h, cuv, out, idx_v, rows_v, sem):
    c = lax.axis_index("c")
    s = lax.axis_index("s")
    base = s * 256
    pltpu.sync_copy(cuv.at[c, pl.ds(base, 256)], idx_v)
    pltpu.async_copy(h.at[idx_v], rows_v, sem).wait()
    pltpu.sync_copy(rows_v, out.at[c, pl.ds(base, 256), :])


def _encode_body(kid, other, d0, d1, ktab, w, z4_ref, nrm_ref):
    n = lax.rsqrt(d0[...] + d1[...] + 1.0)
    oh = (kid[...] == lax.broadcasted_iota(_i32, (R, 8), 1)).astype(_f32)
    emb = jnp.dot(oh, ktab[...], preferred_element_type=_f32,
                  precision=lax.Precision.HIGHEST)
    x = jnp.concatenate([emb, other[...], jnp.zeros((R, 3), _f32)], axis=1)
    xw = jnp.dot(x, w[...], preferred_element_type=_f32)
    row = (pl.program_id(0) * R
           + lax.broadcasted_iota(_i32, (R, 1), 0))
    z = xw * (n * (row < N).astype(_f32))
    z4_ref[...] = jnp.stack([z[:, :16], z[:, 16:32], z[:, 32:48], z[:, 48:]],
                            axis=0)
    nrm_ref[...] = n


def _tc_encode(kid2, other, d0, d1, ktab8, w0p):
    return pl.pallas_call(
        _encode_body,
        grid=(NG,),
        in_specs=[
            pl.BlockSpec((R, 1), lambda i: (i, 0)),
            pl.BlockSpec((R, 5), lambda i: (i, 0)),
            pl.BlockSpec((R, 1), lambda i: (i, 0)),
            pl.BlockSpec((R, 1), lambda i: (i, 0)),
            pl.BlockSpec((8, 8), lambda i: (0, 0)),
            pl.BlockSpec((16, H), lambda i: (0, 0)),
        ],
        out_specs=[
            pl.BlockSpec((4, R, 16), lambda i: (0, i, 0)),
            pl.BlockSpec((R, 1), lambda i: (i, 0)),
        ],
        out_shape=[
            jax.ShapeDtypeStruct((4, NPAD, 16), _f32),
            jax.ShapeDtypeStruct((NPAD, 1), _f32),
        ],
    )(kid2, other, d0, d1, ktab8, w0p)


def _combine0_body(s2, nrm, w, b, z4_ref):
    n = nrm[...]
    sblk = s2[...]
    agg = n * (sblk[0] + sblk[1])
    h = jnp.maximum(jnp.dot(agg, w[...], preferred_element_type=_f32, precision=lax.Precision.HIGHEST) + b[...], 0.0)
    row = pl.program_id(0) * R + lax.broadcasted_iota(_i32, (R, 1), 0)
    z = h * (n * (row < N).astype(_f32))
    z4_ref[...] = jnp.stack([z[:, :16], z[:, 16:32], z[:, 32:48], z[:, 48:]],
                            axis=0)


def _tc_combine0(s2, nrm, w0p, b0):
    return pl.pallas_call(
        _combine0_body,
        grid=(NG,),
        in_specs=[
            pl.BlockSpec((2, R, 16), lambda i: (0, i, 0)),
            pl.BlockSpec((R, 1), lambda i: (i, 0)),
            pl.BlockSpec((16, H), lambda i: (0, 0)),
            pl.BlockSpec((1, H), lambda i: (0, 0)),
        ],
        out_specs=pl.BlockSpec((4, R, 16), lambda i: (0, i, 0)),
        out_shape=jax.ShapeDtypeStruct((4, NPAD, 16), _f32),
    )(s2, nrm, w0p, b0)


def _combine_body(s4, nrm, b, w, z4_ref):
    n = nrm[...]
    sblk = s4[...]
    ss = jnp.concatenate([sblk[0], sblk[1], sblk[2], sblk[3]], axis=1)
    h = jnp.maximum(n * ss + b[...], 0.0)
    hw = jnp.dot(h, w[...], preferred_element_type=_f32)
    row = pl.program_id(0) * R + lax.broadcasted_iota(_i32, (R, 1), 0)
    zo = hw * (n * (row < N).astype(_f32))
    z4_ref[...] = jnp.stack([zo[:, :16], zo[:, 16:32], zo[:, 32:48],
                             zo[:, 48:]], axis=0)


def _tc_combine(s4, nrm, b_prev, w_next):
    return pl.pallas_call(
        _combine_body,
        grid=(NG,),
        in_specs=[
            pl.BlockSpec((4, R, 16), lambda i: (0, i, 0)),
            pl.BlockSpec((R, 1), lambda i: (i, 0)),
            pl.BlockSpec((1, H), lambda i: (0, 0)),
            pl.BlockSpec((H, H), lambda i: (0, 0)),
        ],
        out_specs=pl.BlockSpec((4, R, 16), lambda i: (0, i, 0)),
        out_shape=jax.ShapeDtypeStruct((4, NPAD, 16), _f32),
    )(s4, nrm, b_prev, w_next)


def _final_body(s4, nrm, b, h_ref, acc_ref):
    n = nrm[...]
    sblk = s4[...]
    ss = jnp.concatenate([sblk[0], sblk[1], sblk[2], sblk[3]], axis=1)
    h = jnp.maximum(n * ss + b[...], 0.0)
    h_ref[...] = h

    @pl.when(pl.program_id(0) == 0)
    def _():
        acc_ref[...] = jnp.zeros((8, H), _f32)

    row = pl.program_id(0) * R + lax.broadcasted_iota(_i32, (R, 1), 0)
    hm = h * (row < N).astype(_f32)
    acc_ref[...] += jnp.sum(hm.reshape(R // 8, 8, H), axis=0)


def _tc_final(s4, nrm, b):
    return pl.pallas_call(
        _final_body,
        grid=(NG,),
        in_specs=[
            pl.BlockSpec((4, R, 16), lambda i: (0, i, 0)),
            pl.BlockSpec((R, 1), lambda i: (i, 0)),
            pl.BlockSpec((1, H), lambda i: (0, 0)),
        ],
        out_specs=[
            pl.BlockSpec((R, H), lambda i: (i, 0)),
            pl.BlockSpec((8, H), lambda i: (0, 0)),
        ],
        out_shape=[
            jax.ShapeDtypeStruct((NPAD, H), _f32),
            jax.ShapeDtypeStruct((8, H), _f32),
        ],
    )(s4, nrm, b)


def _score_body(u, v, acc, wg, bg, wc1, bc1, w2, bc2, out_ref):
    tot = jnp.sum(acc[...], axis=0, keepdims=True) * (1.0 / N)
    g = jnp.dot(tot, wg[...], preferred_element_type=_f32) + bg[...]
    feat = jnp.concatenate(
        [u[...], v[...], jnp.broadcast_to(g, (C, H))], axis=1)
    hid = jnp.maximum(jnp.dot(feat, wc1[...], preferred_element_type=_f32)
                      + bc1[...], 0.0)
    lg = jnp.dot(hid, w2[...], preferred_element_type=_f32) + bc2[...]
    out_ref[...] = lg


def _tc_score(u, v, acc, wg, bg, wc1, bc1, w2, bc2):
    whole = lambda shp: pl.BlockSpec(shp, lambda: (0, 0))
    return pl.pallas_call(
        _score_body,
        in_specs=[
            whole((C, H)), whole((C, H)), whole((8, H)),
            whole((H, H)), whole((1, H)),
            whole((3 * H, H)), whole((1, H)),
            whole((H, 1)), whole((1, 1)),
        ],
        out_specs=whole((C, 1)),
        out_shape=jax.ShapeDtypeStruct((C, 1), _f32),
    )(u, v, acc, wg, bg, wc1, bc1, w2, bc2)


def kernel(kind_ids, other_feats, edge_index, cand_u, cand_v, kind_table,
           W0, b0, W1, b1, W2, b2, Wg, bg, Wc1, bc1, Wc2, bc2):
    ei = edge_index.astype(_i32)
    loop = jnp.arange(N, dtype=_i32)
    nd = EX - E - N
    src = jnp.concatenate([ei[0], loop, jnp.full((nd,), N, _i32)])
    dst = jnp.concatenate([ei[1], loop, jnp.arange(nd, dtype=_i32) % N])
    deg2 = _sc_degree(dst)
    ktab8 = jnp.concatenate([kind_table, jnp.zeros((2, 8), _f32)], axis=0)
    kid_p = jnp.pad(kind_ids.astype(_i32), (0, NPAD - N)).reshape(NPAD, 1)
    of_p = jnp.pad(other_feats, ((0, NPAD - N), (0, 0)))
    w0p = jnp.concatenate([W0, jnp.zeros((3, H), _f32)], axis=0)
    z0, nrm = _tc_encode(kid_p, of_p, deg2[0].reshape(NPAD, 1),
                         deg2[1].reshape(NPAD, 1), ktab8, w0p)
    s0 = _sc_segsum64(src, dst, z0)
    z1 = _tc_combine(s0, nrm, b0.reshape(1, H), W1)
    s1 = _sc_segsum64(src, dst, z1)
    z2 = _tc_combine(s1, nrm, b1.reshape(1, H), W2)
    s2 = _sc_segsum64(src, dst, z2)
    h3, acc = _tc_final(s2, nrm, b2.reshape(1, H))
    cuv = jnp.stack([cand_u.astype(_i32), cand_v.astype(_i32)], axis=0)
    uvr = _sc_cand_gather(h3, cuv)
    ur, vr = uvr[0], uvr[1]
    lg = _tc_score(ur, vr, acc, Wg, bg.reshape(1, H), Wc1,
                   bc1.reshape(1, H), Wc2, bc2.reshape(1, 1))
    return lg.reshape(C)


# speedup vs baseline: 1.0013x; 1.0013x over previous
import functools

import jax
import jax.numpy as jnp
from jax import lax
from jax.experimental import pallas as pl
from jax.experimental.pallas import tpu as pltpu
from jax.experimental.pallas import tpu_sc as plsc

N = 50000
E = 800000
C = 4096
H = 64
NPAD = 50048
RPT = 3128
EB = 1000
EX = 896000
R = 3128
NG = NPAD // R

_mesh = plsc.VectorSubcoreMesh(core_axis_name="c", subcore_axis_name="s")

_f32 = jnp.float32
_i32 = jnp.int32


def _fill_const_2d(buf, nrows, width, val):
    vec = jnp.full((16,), val, _f32)

    def body(i, carry):
        for w0 in range(0, width, 16):
            buf[i, pl.ds(w0, 16)] = vec
        return carry

    lax.fori_loop(0, nrows, body, 0)


def _fill_const_1d(buf, n, val):
    vec = jnp.full((16,), val, _f32)

    def body(i, carry):
        buf[pl.ds(i * 16, 16)] = vec
        return carry

    lax.fori_loop(0, n // 16, body, 0)
    if n % 16:
        buf[pl.ds(n - 16, 16)] = vec


def _zero_rows_2d(agg, zbuf, row0):
    for off, sz in _CHUNKS512:
        pltpu.sync_copy(zbuf.at[pl.ds(0, sz), :], agg.at[pl.ds(row0 + off, sz), :])


def _zero_rows_1d(agg, zbuf, row0):
    for off in (0, 1024, 2048):
        pltpu.sync_copy(zbuf.at[pl.ds(0, 1024)], agg.at[pl.ds(row0 + off, 1024)])
    pltpu.sync_copy(zbuf.at[pl.ds(0, 56)], agg.at[pl.ds(row0 + 3072, 56)])


_CHUNKS = ((0, 1024), (1024, 1024), (2048, 1024), (3072, 56))
_CHUNKS512 = ((0, 512), (512, 512), (1024, 512), (1536, 512), (2048, 512),
              (2560, 512), (3072, 56))


def _writeout_2d(agg, buf, out, c, row0):
    for off, sz in _CHUNKS:
        pltpu.sync_copy(agg.at[pl.ds(row0 + off, sz), :], buf.at[pl.ds(0, sz), :])
        pltpu.sync_copy(buf.at[pl.ds(0, sz), :], out.at[c, pl.ds(row0 + off, sz), :])


def _issue_idx(src, dst, sv, dv, sem, b):
    pltpu.async_copy(src.at[pl.ds(b, EB)], sv, sem)
    pltpu.async_copy(dst.at[pl.ds(b, EB)], dv, sem)


def _wait_idx(src, sv, dv, sem):
    pltpu.make_async_copy(src.at[pl.ds(0, EB)], sv, sem).wait()
    pltpu.make_async_copy(src.at[pl.ds(0, EB)], dv, sem).wait()


def _segsum_edges_pipe(src, dst, z, agg, sv, dv, rv, si, sg, ss, base, nb):
    _issue_idx(src, dst, sv[0], dv[0], si[0], base)
    _issue_idx(src, dst, sv[1], dv[1], si[1], base + EB)
    _wait_idx(src, sv[0], dv[0], si[0])
    pltpu.async_copy(z.at[sv[0]], rv[0], sg[0])

    def quad(k, carry):
        for b4 in range(4):
            g = 4 * k + b4
            rb = b4 % 2
            rb2 = 1 - rb
            j1 = (b4 + 1) % 4
            j2 = (b4 + 2) % 4

            @pl.when(g >= 1)
            def _():
                pltpu.make_async_copy(rv[rb2], agg.at[dv[0]], ss[rb2]).wait()

            @pl.when(g + 1 < nb)
            def _():
                _wait_idx(src, sv[j1], dv[j1], si[j1])
                pltpu.async_copy(z.at[sv[j1]], rv[rb2], sg[rb2])

            pltpu.make_async_copy(z.at[sv[0]], rv[rb], sg[rb]).wait()

            @pl.when(g < nb - 1)
            def _():
                pltpu.async_copy(rv[rb], agg.at[dv[b4]], ss[rb], add=True)

            @pl.when(g == nb - 1)
            def _():
                pltpu.sync_copy(rv[rb], agg.at[dv[b4]], add=True)

            @pl.when(g + 2 < nb)
            def _():
                _issue_idx(src, dst, sv[j2], dv[j2], si[j2],
                           base + (g + 2) * EB)
        return carry

    lax.fori_loop(0, nb // 4, quad, 0)


@functools.partial(
    pl.kernel,
    out_type=jax.ShapeDtypeStruct((2, NPAD), _f32),
    mesh=_mesh,
    compiler_params=pltpu.CompilerParams(use_tc_tiling_on_sc=False),
    scratch_types=[
        pltpu.VMEM((EB,), _i32),
        pltpu.VMEM((EB,), _f32),
        pltpu.VMEM((1024,), _f32),
        pltpu.VMEM_SHARED((NPAD,), _f32),
    ],
)
def _sc_degree(dst, out, dst_v, ones_v, zbuf, deg_sh):
    c = lax.axis_index("c")
    s = lax.axis_index("s")
    row0 = s * RPT
    _fill_const_1d(zbuf, 1024, 0.0)
    _fill_const_1d(ones_v, EB, 1.0)
    _zero_rows_1d(deg_sh, zbuf, row0)
    plsc.subcore_barrier()

    base = c * 400000 + s * 25000

    def body(i, carry):
        pltpu.sync_copy(dst.at[pl.ds(base + i * EB, EB)], dst_v)
        pltpu.sync_copy(ones_v.at[pl.ds(0, EB)], deg_sh.at[dst_v], add=True)
        return carry

    lax.fori_loop(0, 25, body, 0)
    plsc.subcore_barrier()

    for off, sz in _CHUNKS:
        pltpu.sync_copy(deg_sh.at[pl.ds(row0 + off, sz)], zbuf.at[pl.ds(0, sz)])
        pltpu.sync_copy(zbuf.at[pl.ds(0, sz)], out.at[c, pl.ds(row0 + off, sz)])


@functools.partial(
    pl.kernel,
    out_type=jax.ShapeDtypeStruct((4, NPAD, 16), _f32),
    mesh=_mesh,
    compiler_params=pltpu.CompilerParams(use_tc_tiling_on_sc=False),
    scratch_types=[
        pltpu.VMEM((EB,), _i32), pltpu.VMEM((EB,), _i32),
        pltpu.VMEM((EB,), _i32), pltpu.VMEM((EB,), _i32),
        pltpu.VMEM((EB,), _i32), pltpu.VMEM((EB,), _i32),
        pltpu.VMEM((EB,), _i32), pltpu.VMEM((EB,), _i32),
        pltpu.VMEM((EB, 16), _f32), pltpu.VMEM((EB, 16), _f32),
        pltpu.VMEM((512, 16), _f32),
        pltpu.VMEM((512, 16), _f32),
        pltpu.VMEM_SHARED((NPAD, 16), _f32),
        pltpu.SemaphoreType.DMA, pltpu.SemaphoreType.DMA,
        pltpu.SemaphoreType.DMA, pltpu.SemaphoreType.DMA,
        pltpu.SemaphoreType.DMA, pltpu.SemaphoreType.DMA,
        pltpu.SemaphoreType.DMA, pltpu.SemaphoreType.DMA,
    ],
)
def _sc_segsum64(src, dst, z4, out, sv0, sv1, sv2, sv3, dv0, dv1, dv2, dv3,
                 rv0, rv1, zbuf, wbuf, agg,
                 si0, si1, si2, si3, sg0, sg1, ss0, ss1):
    c = lax.axis_index("c")
    s = lax.axis_index("s")
    row0 = s * RPT
    sv = [sv0, sv1, sv2, sv3]
    dv = [dv0, dv1, dv2, dv3]
    rv = [rv0, rv1]
    si = [si0, si1, si2, si3]
    sg = [sg0, sg1]
    ss = [ss0, ss1]
    _fill_const_2d(zbuf, 512, 16, 0.0)
    _zero_rows_2d(agg, zbuf, row0)

    base = s * (EX // 16)
    nb = EX // 16 // EB
    for p in range(2):
        chunk = 2 * c + p
        plsc.subcore_barrier()
        _segsum_edges_pipe(src, dst, z4.at[chunk], agg, sv, dv, rv,
                           si, sg, ss, base, nb)
        plsc.subcore_barrier()
        for off, sz in _CHUNKS512:
            pltpu.sync_copy(agg.at[pl.ds(row0 + off, sz), :],
                            wbuf.at[pl.ds(0, sz), :])
            pltpu.sync_copy(wbuf.at[pl.ds(0, sz), :],
                            out.at[chunk, pl.ds(row0 + off, sz), :])
        if p == 0:
            _zero_rows_2d(agg, zbuf, row0)


@functools.partial(
    pl.kernel,
    out_type=jax.ShapeDtypeStruct((2, C, H), _f32),
    mesh=_mesh,
    compiler_params=pltpu.CompilerParams(use_tc_tiling_on_sc=False),
    scratch_types=[
        pltpu.VMEM((256,), _i32),
        pltpu.VMEM((256, H), _f32),
        pltpu.SemaphoreType.DMA,
    ],
)
def _sc_cand_gather(h, cuv, out, idx_v, rows_v, sem):
    c = lax.axis_index("c")
    s = lax.axis_index("s")
    base = s * 256
    pltpu.sync_copy(cuv.at[c, pl.ds(base, 256)], idx_v)
    pltpu.async_copy(h.at[idx_v], rows_v, sem).wait()
    pltpu.sync_copy(rows_v, out.at[c, pl.ds(base, 256), :])


def _encode_body(kid, other, d0, d1, ktab, w, z4_ref, nrm_ref):
    n = lax.rsqrt(d0[...] + d1[...] + 1.0)
    oh = (kid[...] == lax.broadcasted_iota(_i32, (R, 8), 1)).astype(_f32)
    emb = jnp.dot(oh, ktab[...], preferred_element_type=_f32,
                  precision=lax.Precision.HIGHEST)
    x = jnp.concatenate([emb, other[...], jnp.zeros((R, 3), _f32)], axis=1)
    xw = jnp.dot(x, w[...], preferred_element_type=_f32)
    row = (pl.program_id(0) * R
           + lax.broadcasted_iota(_i32, (R, 1), 0))
    z = xw * (n * (row < N).astype(_f32))
    z4_ref[...] = jnp.stack([z[:, :16], z[:, 16:32], z[:, 32:48], z[:, 48:]],
                            axis=0)
    nrm_ref[...] = n


def _tc_encode(kid2, other, d0, d1, ktab8, w0p):
    return pl.pallas_call(
        _encode_body,
        grid=(NG,),
        in_specs=[
            pl.BlockSpec((R, 1), lambda i: (i, 0)),
            pl.BlockSpec((R, 5), lambda i: (i, 0)),
            pl.BlockSpec((R, 1), lambda i: (i, 0)),
            pl.BlockSpec((R, 1), lambda i: (i, 0)),
            pl.BlockSpec((8, 8), lambda i: (0, 0)),
            pl.BlockSpec((16, H), lambda i: (0, 0)),
        ],
        out_specs=[
            pl.BlockSpec((4, R, 16), lambda i: (0, i, 0)),
            pl.BlockSpec((R, 1), lambda i: (i, 0)),
        ],
        out_shape=[
            jax.ShapeDtypeStruct((4, NPAD, 16), _f32),
            jax.ShapeDtypeStruct((NPAD, 1), _f32),
        ],
    )(kid2, other, d0, d1, ktab8, w0p)


def _combine0_body(s2, nrm, w, b, z4_ref):
    n = nrm[...]
    sblk = s2[...]
    agg = n * (sblk[0] + sblk[1])
    h = jnp.maximum(jnp.dot(agg, w[...], preferred_element_type=_f32, precision=lax.Precision.HIGHEST) + b[...], 0.0)
    row = pl.program_id(0) * R + lax.broadcasted_iota(_i32, (R, 1), 0)
    z = h * (n * (row < N).astype(_f32))
    z4_ref[...] = jnp.stack([z[:, :16], z[:, 16:32], z[:, 32:48], z[:, 48:]],
                            axis=0)


def _tc_combine0(s2, nrm, w0p, b0):
    return pl.pallas_call(
        _combine0_body,
        grid=(NG,),
        in_specs=[
            pl.BlockSpec((2, R, 16), lambda i: (0, i, 0)),
            pl.BlockSpec((R, 1), lambda i: (i, 0)),
            pl.BlockSpec((16, H), lambda i: (0, 0)),
            pl.BlockSpec((1, H), lambda i: (0, 0)),
        ],
        out_specs=pl.BlockSpec((4, R, 16), lambda i: (0, i, 0)),
        out_shape=jax.ShapeDtypeStruct((4, NPAD, 16), _f32),
    )(s2, nrm, w0p, b0)


def _combine_body(s4, nrm, b, w, z4_ref):
    n = nrm[...]
    sblk = s4[...]
    ss = jnp.concatenate([sblk[0], sblk[1], sblk[2], sblk[3]], axis=1)
    h = jnp.maximum(n * ss + b[...], 0.0)
    hw = jnp.dot(h, w[...], preferred_element_type=_f32)
    row = pl.program_id(0) * R + lax.broadcasted_iota(_i32, (R, 1), 0)
    zo = hw * (n * (row < N).astype(_f32))
    z4_ref[...] = jnp.stack([zo[:, :16], zo[:, 16:32], zo[:, 32:48],
                             zo[:, 48:]], axis=0)


def _tc_combine(s4, nrm, b_prev, w_next):
    return pl.pallas_call(
        _combine_body,
        grid=(NG,),
        in_specs=[
            pl.BlockSpec((4, R, 16), lambda i: (0, i, 0)),
            pl.BlockSpec((R, 1), lambda i: (i, 0)),
            pl.BlockSpec((1, H), lambda i: (0, 0)),
            pl.BlockSpec((H, H), lambda i: (0, 0)),
        ],
        out_specs=pl.BlockSpec((4, R, 16), lambda i: (0, i, 0)),
        out_shape=jax.ShapeDtypeStruct((4, NPAD, 16), _f32),
    )(s4, nrm, b_prev, w_next)


def _final_body(s4, nrm, b, h_ref, acc_ref):
    n = nrm[...]
    sblk = s4[...]
    ss = jnp.concatenate([sblk[0], sblk[1], sblk[2], sblk[3]], axis=1)
    h = jnp.maximum(n * ss + b[...], 0.0)
    h_ref[...] = h

    @pl.when(pl.program_id(0) == 0)
    def _():
        acc_ref[...] = jnp.zeros((8, H), _f32)

    row = pl.program_id(0) * R + lax.broadcasted_iota(_i32, (R, 1), 0)
    hm = h * (row < N).astype(_f32)
    acc_ref[...] += jnp.sum(hm.reshape(R // 8, 8, H), axis=0)


def _tc_final(s4, nrm, b):
    return pl.pallas_call(
        _final_body,
        grid=(NG,),
        in_specs=[
            pl.BlockSpec((4, R, 16), lambda i: (0, i, 0)),
            pl.BlockSpec((R, 1), lambda i: (i, 0)),
            pl.BlockSpec((1, H), lambda i: (0, 0)),
        ],
        out_specs=[
            pl.BlockSpec((R, H), lambda i: (i, 0)),
            pl.BlockSpec((8, H), lambda i: (0, 0)),
        ],
        out_shape=[
            jax.ShapeDtypeStruct((NPAD, H), _f32),
            jax.ShapeDtypeStruct((8, H), _f32),
        ],
    )(s4, nrm, b)


def _score_body(u, v, acc, wg, bg, wc1, bc1, w2, bc2, out_ref):
    tot = jnp.sum(acc[...], axis=0, keepdims=True) * (1.0 / N)
    g = jnp.dot(tot, wg[...], preferred_element_type=_f32) + bg[...]
    feat = jnp.concatenate(
        [u[...], v[...], jnp.broadcast_to(g, (C, H))], axis=1)
    hid = jnp.maximum(jnp.dot(feat, wc1[...], preferred_element_type=_f32)
                      + bc1[...], 0.0)
    lg = jnp.dot(hid, w2[...], preferred_element_type=_f32) + bc2[...]
    out_ref[...] = lg


def _tc_score(u, v, acc, wg, bg, wc1, bc1, w2, bc2):
    whole = lambda shp: pl.BlockSpec(shp, lambda: (0, 0))
    return pl.pallas_call(
        _score_body,
        in_specs=[
            whole((C, H)), whole((C, H)), whole((8, H)),
            whole((H, H)), whole((1, H)),
            whole((3 * H, H)), whole((1, H)),
            whole((H, 1)), whole((1, 1)),
        ],
        out_specs=whole((C, 1)),
        out_shape=jax.ShapeDtypeStruct((C, 1), _f32),
    )(u, v, acc, wg, bg, wc1, bc1, w2, bc2)


def kernel(kind_ids, other_feats, edge_index, cand_u, cand_v, kind_table,
           W0, b0, W1, b1, W2, b2, Wg, bg, Wc1, bc1, Wc2, bc2):
    ei = edge_index.astype(_i32)
    loop = jnp.arange(N, dtype=_i32)
    nd = EX - E - N
    src = jnp.concatenate([ei[0], loop, jnp.full((nd,), N, _i32)])
    dst = jnp.concatenate([ei[1], loop, jnp.arange(nd, dtype=_i32) % N])
    deg2 = _sc_degree(dst)
    ktab8 = jnp.concatenate([kind_table, jnp.zeros((2, 8), _f32)], axis=0)
    kid_p = jnp.pad(kind_ids.astype(_i32), (0, NPAD - N)).reshape(NPAD, 1)
    of_p = jnp.pad(other_feats, ((0, NPAD - N), (0, 0)))
    w0p = jnp.concatenate([W0, jnp.zeros((3, H), _f32)], axis=0)
    z0, nrm = _tc_encode(kid_p, of_p, deg2[0].reshape(NPAD, 1),
                         deg2[1].reshape(NPAD, 1), ktab8, w0p)
    s0 = _sc_segsum64(src, dst, z0)
    z1 = _tc_combine(s0, nrm, b0.reshape(1, H), W1)
    s1 = _sc_segsum64(src, dst, z1)
    z2 = _tc_combine(s1, nrm, b1.reshape(1, H), W2)
    s2 = _sc_segsum64(src, dst, z2)
    h3, acc = _tc_final(s2, nrm, b2.reshape(1, H))
    cuv = jnp.stack([cand_u.astype(_i32), cand_v.astype(_i32)], axis=0)
    uvr = _sc_cand_gather(h3, cuv)
    ur, vr = uvr[0], uvr[1]
    lg = _tc_score(ur, vr, acc, Wg, bg.reshape(1, H), Wc1,
                   bc1.reshape(1, H), Wc2, bc2.reshape(1, 1))
    return lg.reshape(C)


# speedup vs baseline: 1.0016x; 1.0003x over previous
import functools

import jax
import jax.numpy as jnp
from jax import lax
from jax.experimental import pallas as pl
from jax.experimental.pallas import tpu as pltpu
from jax.experimental.pallas import tpu_sc as plsc

N = 50000
E = 800000
C = 4096
H = 64
NPAD = 50048
RPT = 3128
EB = 1000
DB = 1000
EX = 896000
R = 3128
NG = NPAD // R

_mesh = plsc.VectorSubcoreMesh(core_axis_name="c", subcore_axis_name="s")

_f32 = jnp.float32
_i32 = jnp.int32


def _fill_const_2d(buf, nrows, width, val):
    vec = jnp.full((16,), val, _f32)

    def body(i, carry):
        for w0 in range(0, width, 16):
            buf[i, pl.ds(w0, 16)] = vec
        return carry

    lax.fori_loop(0, nrows, body, 0)


def _fill_const_1d(buf, n, val):
    vec = jnp.full((16,), val, _f32)

    def body(i, carry):
        buf[pl.ds(i * 16, 16)] = vec
        return carry

    lax.fori_loop(0, n // 16, body, 0)
    if n % 16:
        buf[pl.ds(n - 16, 16)] = vec


def _zero_rows_2d(agg, zbuf, row0):
    for off, sz in _CHUNKS512:
        pltpu.sync_copy(zbuf.at[pl.ds(0, sz), :], agg.at[pl.ds(row0 + off, sz), :])


def _zero_rows_1d(agg, zbuf, row0):
    for off in (0, 1024, 2048):
        pltpu.sync_copy(zbuf.at[pl.ds(0, 1024)], agg.at[pl.ds(row0 + off, 1024)])
    pltpu.sync_copy(zbuf.at[pl.ds(0, 56)], agg.at[pl.ds(row0 + 3072, 56)])


_CHUNKS = ((0, 1024), (1024, 1024), (2048, 1024), (3072, 56))
_CHUNKS512 = ((0, 512), (512, 512), (1024, 512), (1536, 512), (2048, 512),
              (2560, 512), (3072, 56))


def _writeout_2d(agg, buf, out, c, row0):
    for off, sz in _CHUNKS:
        pltpu.sync_copy(agg.at[pl.ds(row0 + off, sz), :], buf.at[pl.ds(0, sz), :])
        pltpu.sync_copy(buf.at[pl.ds(0, sz), :], out.at[c, pl.ds(row0 + off, sz), :])


def _issue_idx(src, dst, sv, dv, sem, b):
    pltpu.async_copy(src.at[pl.ds(b, EB)], sv, sem)
    pltpu.async_copy(dst.at[pl.ds(b, EB)], dv, sem)


def _wait_idx(src, sv, dv, sem):
    pltpu.make_async_copy(src.at[pl.ds(0, EB)], sv, sem).wait()
    pltpu.make_async_copy(src.at[pl.ds(0, EB)], dv, sem).wait()


def _segsum_edges_pipe(src, dst, z, agg, sv, dv, rv, si, sg, ss, base, nb):
    _issue_idx(src, dst, sv[0], dv[0], si[0], base)
    _issue_idx(src, dst, sv[1], dv[1], si[1], base + EB)
    _wait_idx(src, sv[0], dv[0], si[0])
    pltpu.async_copy(z.at[sv[0]], rv[0], sg[0])

    def quad(k, carry):
        for b4 in range(4):
            g = 4 * k + b4
            rb = b4 % 2
            rb2 = 1 - rb
            j1 = (b4 + 1) % 4
            j2 = (b4 + 2) % 4

            @pl.when(g >= 1)
            def _():
                pltpu.make_async_copy(rv[rb2], agg.at[dv[0]], ss[rb2]).wait()

            @pl.when(g + 1 < nb)
            def _():
                _wait_idx(src, sv[j1], dv[j1], si[j1])
                pltpu.async_copy(z.at[sv[j1]], rv[rb2], sg[rb2])

            pltpu.make_async_copy(z.at[sv[0]], rv[rb], sg[rb]).wait()

            @pl.when(g < nb - 1)
            def _():
                pltpu.async_copy(rv[rb], agg.at[dv[b4]], ss[rb], add=True)

            @pl.when(g == nb - 1)
            def _():
                pltpu.sync_copy(rv[rb], agg.at[dv[b4]], add=True)

            @pl.when(g + 2 < nb)
            def _():
                _issue_idx(src, dst, sv[j2], dv[j2], si[j2],
                           base + (g + 2) * EB)
        return carry

    lax.fori_loop(0, nb // 4, quad, 0)


@functools.partial(
    pl.kernel,
    out_type=jax.ShapeDtypeStruct((2, NPAD), _f32),
    mesh=_mesh,
    compiler_params=pltpu.CompilerParams(use_tc_tiling_on_sc=False),
    scratch_types=[
        pltpu.VMEM((DB,), _i32),
        pltpu.VMEM((DB,), _f32),
        pltpu.VMEM((1024,), _f32),
        pltpu.VMEM_SHARED((NPAD,), _f32),
    ],
)
def _sc_degree(dst, out, dst_v, ones_v, zbuf, deg_sh):
    c = lax.axis_index("c")
    s = lax.axis_index("s")
    row0 = s * RPT
    _fill_const_1d(zbuf, 1024, 0.0)
    _fill_const_1d(ones_v, DB, 1.0)
    _zero_rows_1d(deg_sh, zbuf, row0)
    plsc.subcore_barrier()

    base = c * 400000 + s * 25000

    def body(i, carry):
        pltpu.sync_copy(dst.at[pl.ds(base + i * DB, DB)], dst_v)
        pltpu.sync_copy(ones_v.at[pl.ds(0, DB)], deg_sh.at[dst_v], add=True)
        return carry

    lax.fori_loop(0, 25, body, 0)
    plsc.subcore_barrier()

    for off, sz in _CHUNKS:
        pltpu.sync_copy(deg_sh.at[pl.ds(row0 + off, sz)], zbuf.at[pl.ds(0, sz)])
        pltpu.sync_copy(zbuf.at[pl.ds(0, sz)], out.at[c, pl.ds(row0 + off, sz)])


@functools.partial(
    pl.kernel,
    out_type=jax.ShapeDtypeStruct((4, NPAD, 16), _f32),
    mesh=_mesh,
    compiler_params=pltpu.CompilerParams(use_tc_tiling_on_sc=False),
    scratch_types=[
        pltpu.VMEM((EB,), _i32), pltpu.VMEM((EB,), _i32),
        pltpu.VMEM((EB,), _i32), pltpu.VMEM((EB,), _i32),
        pltpu.VMEM((EB,), _i32), pltpu.VMEM((EB,), _i32),
        pltpu.VMEM((EB,), _i32), pltpu.VMEM((EB,), _i32),
        pltpu.VMEM((EB, 16), _f32), pltpu.VMEM((EB, 16), _f32),
        pltpu.VMEM((512, 16), _f32),
        pltpu.VMEM((512, 16), _f32),
        pltpu.VMEM_SHARED((NPAD, 16), _f32),
        pltpu.SemaphoreType.DMA, pltpu.SemaphoreType.DMA,
        pltpu.SemaphoreType.DMA, pltpu.SemaphoreType.DMA,
        pltpu.SemaphoreType.DMA, pltpu.SemaphoreType.DMA,
        pltpu.SemaphoreType.DMA, pltpu.SemaphoreType.DMA,
    ],
)
def _sc_segsum64(src, dst, z4, out, sv0, sv1, sv2, sv3, dv0, dv1, dv2, dv3,
                 rv0, rv1, zbuf, wbuf, agg,
                 si0, si1, si2, si3, sg0, sg1, ss0, ss1):
    c = lax.axis_index("c")
    s = lax.axis_index("s")
    row0 = s * RPT
    sv = [sv0, sv1, sv2, sv3]
    dv = [dv0, dv1, dv2, dv3]
    rv = [rv0, rv1]
    si = [si0, si1, si2, si3]
    sg = [sg0, sg1]
    ss = [ss0, ss1]
    _fill_const_2d(zbuf, 512, 16, 0.0)
    _zero_rows_2d(agg, zbuf, row0)

    base = s * (EX // 16)
    nb = EX // 16 // EB
    for p in range(2):
        chunk = 2 * c + p
        plsc.subcore_barrier()
        _segsum_edges_pipe(src, dst, z4.at[chunk], agg, sv, dv, rv,
                           si, sg, ss, base, nb)
        plsc.subcore_barrier()
        for off, sz in _CHUNKS512:
            pltpu.sync_copy(agg.at[pl.ds(row0 + off, sz), :],
                            wbuf.at[pl.ds(0, sz), :])
            pltpu.sync_copy(wbuf.at[pl.ds(0, sz), :],
                            out.at[chunk, pl.ds(row0 + off, sz), :])
        if p == 0:
            _zero_rows_2d(agg, zbuf, row0)


@functools.partial(
    pl.kernel,
    out_type=jax.ShapeDtypeStruct((2, C, H), _f32),
    mesh=_mesh,
    compiler_params=pltpu.CompilerParams(use_tc_tiling_on_sc=False),
    scratch_types=[
        pltpu.VMEM((256,), _i32),
        pltpu.VMEM((256, H), _f32),
        pltpu.SemaphoreType.DMA,
    ],
)
def _sc_cand_gather(h, cuv, out, idx_v, rows_v, sem):
    c = lax.axis_index("c")
    s = lax.axis_index("s")
    base = s * 256
    pltpu.sync_copy(cuv.at[c, pl.ds(base, 256)], idx_v)
    pltpu.async_copy(h.at[idx_v], rows_v, sem).wait()
    pltpu.sync_copy(rows_v, out.at[c, pl.ds(base, 256), :])


def _encode_body(kid, other, d0, d1, ktab, w, z4_ref, nrm_ref):
    n = lax.rsqrt(d0[...] + d1[...] + 1.0)
    oh = (kid[...] == lax.broadcasted_iota(_i32, (R, 8), 1)).astype(_f32)
    emb = jnp.dot(oh, ktab[...], preferred_element_type=_f32,
                  precision=lax.Precision.HIGHEST)
    x = jnp.concatenate([emb, other[...], jnp.zeros((R, 3), _f32)], axis=1)
    xw = jnp.dot(x, w[...], preferred_element_type=_f32)
    row = (pl.program_id(0) * R
           + lax.broadcasted_iota(_i32, (R, 1), 0))
    z = xw * (n * (row < N).astype(_f32))
    z4_ref[...] = jnp.stack([z[:, :16], z[:, 16:32], z[:, 32:48], z[:, 48:]],
                            axis=0)
    nrm_ref[...] = n


def _tc_encode(kid2, other, d0, d1, ktab8, w0p):
    return pl.pallas_call(
        _encode_body,
        grid=(NG,),
        in_specs=[
            pl.BlockSpec((R, 1), lambda i: (i, 0)),
            pl.BlockSpec((R, 5), lambda i: (i, 0)),
            pl.BlockSpec((R, 1), lambda i: (i, 0)),
            pl.BlockSpec((R, 1), lambda i: (i, 0)),
            pl.BlockSpec((8, 8), lambda i: (0, 0)),
            pl.BlockSpec((16, H), lambda i: (0, 0)),
        ],
        out_specs=[
            pl.BlockSpec((4, R, 16), lambda i: (0, i, 0)),
            pl.BlockSpec((R, 1), lambda i: (i, 0)),
        ],
        out_shape=[
            jax.ShapeDtypeStruct((4, NPAD, 16), _f32),
            jax.ShapeDtypeStruct((NPAD, 1), _f32),
        ],
    )(kid2, other, d0, d1, ktab8, w0p)


def _combine0_body(s2, nrm, w, b, z4_ref):
    n = nrm[...]
    sblk = s2[...]
    agg = n * (sblk[0] + sblk[1])
    h = jnp.maximum(jnp.dot(agg, w[...], preferred_element_type=_f32, precision=lax.Precision.HIGHEST) + b[...], 0.0)
    row = pl.program_id(0) * R + lax.broadcasted_iota(_i32, (R, 1), 0)
    z = h * (n * (row < N).astype(_f32))
    z4_ref[...] = jnp.stack([z[:, :16], z[:, 16:32], z[:, 32:48], z[:, 48:]],
                            axis=0)


def _tc_combine0(s2, nrm, w0p, b0):
    return pl.pallas_call(
        _combine0_body,
        grid=(NG,),
        in_specs=[
            pl.BlockSpec((2, R, 16), lambda i: (0, i, 0)),
            pl.BlockSpec((R, 1), lambda i: (i, 0)),
            pl.BlockSpec((16, H), lambda i: (0, 0)),
            pl.BlockSpec((1, H), lambda i: (0, 0)),
        ],
        out_specs=pl.BlockSpec((4, R, 16), lambda i: (0, i, 0)),
        out_shape=jax.ShapeDtypeStruct((4, NPAD, 16), _f32),
    )(s2, nrm, w0p, b0)


def _combine_body(s4, nrm, b, w, z4_ref):
    n = nrm[...]
    sblk = s4[...]
    ss = jnp.concatenate([sblk[0], sblk[1], sblk[2], sblk[3]], axis=1)
    h = jnp.maximum(n * ss + b[...], 0.0)
    hw = jnp.dot(h, w[...], preferred_element_type=_f32)
    row = pl.program_id(0) * R + lax.broadcasted_iota(_i32, (R, 1), 0)
    zo = hw * (n * (row < N).astype(_f32))
    z4_ref[...] = jnp.stack([zo[:, :16], zo[:, 16:32], zo[:, 32:48],
                             zo[:, 48:]], axis=0)


def _tc_combine(s4, nrm, b_prev, w_next):
    return pl.pallas_call(
        _combine_body,
        grid=(NG,),
        in_specs=[
            pl.BlockSpec((4, R, 16), lambda i: (0, i, 0)),
            pl.BlockSpec((R, 1), lambda i: (i, 0)),
            pl.BlockSpec((1, H), lambda i: (0, 0)),
            pl.BlockSpec((H, H), lambda i: (0, 0)),
        ],
        out_specs=pl.BlockSpec((4, R, 16), lambda i: (0, i, 0)),
        out_shape=jax.ShapeDtypeStruct((4, NPAD, 16), _f32),
    )(s4, nrm, b_prev, w_next)


def _final_body(s4, nrm, b, h_ref, acc_ref):
    n = nrm[...]
    sblk = s4[...]
    ss = jnp.concatenate([sblk[0], sblk[1], sblk[2], sblk[3]], axis=1)
    h = jnp.maximum(n * ss + b[...], 0.0)
    h_ref[...] = h

    @pl.when(pl.program_id(0) == 0)
    def _():
        acc_ref[...] = jnp.zeros((8, H), _f32)

    row = pl.program_id(0) * R + lax.broadcasted_iota(_i32, (R, 1), 0)
    hm = h * (row < N).astype(_f32)
    acc_ref[...] += jnp.sum(hm.reshape(R // 8, 8, H), axis=0)


def _tc_final(s4, nrm, b):
    return pl.pallas_call(
        _final_body,
        grid=(NG,),
        in_specs=[
            pl.BlockSpec((4, R, 16), lambda i: (0, i, 0)),
            pl.BlockSpec((R, 1), lambda i: (i, 0)),
            pl.BlockSpec((1, H), lambda i: (0, 0)),
        ],
        out_specs=[
            pl.BlockSpec((R, H), lambda i: (i, 0)),
            pl.BlockSpec((8, H), lambda i: (0, 0)),
        ],
        out_shape=[
            jax.ShapeDtypeStruct((NPAD, H), _f32),
            jax.ShapeDtypeStruct((8, H), _f32),
        ],
    )(s4, nrm, b)


def _score_body(u, v, acc, wg, bg, wc1, bc1, w2, bc2, out_ref):
    tot = jnp.sum(acc[...], axis=0, keepdims=True) * (1.0 / N)
    g = jnp.dot(tot, wg[...], preferred_element_type=_f32) + bg[...]
    feat = jnp.concatenate(
        [u[...], v[...], jnp.broadcast_to(g, (C, H))], axis=1)
    hid = jnp.maximum(jnp.dot(feat, wc1[...], preferred_element_type=_f32)
                      + bc1[...], 0.0)
    lg = jnp.dot(hid, w2[...], preferred_element_type=_f32) + bc2[...]
    out_ref[...] = lg


def _tc_score(u, v, acc, wg, bg, wc1, bc1, w2, bc2):
    whole = lambda shp: pl.BlockSpec(shp, lambda: (0, 0))
    return pl.pallas_call(
        _score_body,
        in_specs=[
            whole((C, H)), whole((C, H)), whole((8, H)),
            whole((H, H)), whole((1, H)),
            whole((3 * H, H)), whole((1, H)),
            whole((H, 1)), whole((1, 1)),
        ],
        out_specs=whole((C, 1)),
        out_shape=jax.ShapeDtypeStruct((C, 1), _f32),
    )(u, v, acc, wg, bg, wc1, bc1, w2, bc2)


def kernel(kind_ids, other_feats, edge_index, cand_u, cand_v, kind_table,
           W0, b0, W1, b1, W2, b2, Wg, bg, Wc1, bc1, Wc2, bc2):
    ei = edge_index.astype(_i32)
    loop = jnp.arange(N, dtype=_i32)
    nd = EX - E - N
    src = jnp.concatenate([ei[0], loop, jnp.full((nd,), N, _i32)])
    dst = jnp.concatenate([ei[1], loop, jnp.arange(nd, dtype=_i32) % N])
    deg2 = _sc_degree(dst)
    ktab8 = jnp.concatenate([kind_table, jnp.zeros((2, 8), _f32)], axis=0)
    kid_p = jnp.pad(kind_ids.astype(_i32), (0, NPAD - N)).reshape(NPAD, 1)
    of_p = jnp.pad(other_feats, ((0, NPAD - N), (0, 0)))
    w0p = jnp.concatenate([W0, jnp.zeros((3, H), _f32)], axis=0)
    z0, nrm = _tc_encode(kid_p, of_p, deg2[0].reshape(NPAD, 1),
                         deg2[1].reshape(NPAD, 1), ktab8, w0p)
    s0 = _sc_segsum64(src, dst, z0)
    z1 = _tc_combine(s0, nrm, b0.reshape(1, H), W1)
    s1 = _sc_segsum64(src, dst, z1)
    z2 = _tc_combine(s1, nrm, b1.reshape(1, H), W2)
    s2 = _sc_segsum64(src, dst, z2)
    h3, acc = _tc_final(s2, nrm, b2.reshape(1, H))
    cuv = jnp.stack([cand_u.astype(_i32), cand_v.astype(_i32)], axis=0)
    uvr = _sc_cand_gather(h3, cuv)
    ur, vr = uvr[0], uvr[1]
    lg = _tc_score(ur, vr, acc, Wg, bg.reshape(1, H), Wc1,
                   bc1.reshape(1, H), Wc2, bc2.reshape(1, 1))
    return lg.reshape(C)
